# Initial kernel scaffold; baseline (speedup 1.0000x reference)
import functools

import jax
import jax.numpy as jnp
from jax import lax
from jax.experimental import pallas as pl
from jax.experimental.pallas import tpu as pltpu
from jax.experimental.pallas import tpu_sc as plsc

_N = 10000
_E = 320000
_D = 128
_NC = 2
_NS = 16
_NW = _NC * _NS
_EPT = _E // _NW
_C = 80
_NCHUNK = _EPT // _C
_RPT = _N // _NS

_SC_MESH = plsc.VectorSubcoreMesh(core_axis_name="c", subcore_axis_name="s")


def _deg_kernel(dst_hbm, ones_hbm, zero_hbm, out_hbm, didx, ones_v, acc):
    c = lax.axis_index("c")
    s = lax.axis_index("s")
    r0 = s * _RPT
    pltpu.sync_copy(ones_hbm, ones_v)
    pltpu.sync_copy(zero_hbm.at[pl.ds(0, _RPT)], acc.at[pl.ds(r0, _RPT)])
    plsc.subcore_barrier()
    base = (c * _NS + s) * _EPT

    def body(i, carry):
        e0 = base + i * _C
        pltpu.sync_copy(dst_hbm.at[pl.ds(e0, _C)], didx)
        pltpu.sync_copy(ones_v, acc.at[didx], add=True)
        return carry

    lax.fori_loop(0, _NCHUNK, body, 0)
    plsc.subcore_barrier()
    pltpu.sync_copy(acc.at[pl.ds(r0, _RPT)], out_hbm.at[c, pl.ds(r0, _RPT)])


def _deg_partials(dst, ones16, zeros16):
    return pl.kernel(
        _deg_kernel,
        mesh=_SC_MESH,
        out_type=jax.ShapeDtypeStruct((_NC, _N, 16), jnp.float32),
        scratch_types=[
            pltpu.VMEM((_C,), jnp.int32),
            pltpu.VMEM((_C, 16), jnp.float32),
            pltpu.VMEM_SHARED((_N, 16), jnp.float32),
        ],
    )(dst, ones16, zeros16)


def _agg_kernel(g_hbm, src_hbm, dst_hbm, zero_hbm, out_hbm, sidx, didx, rows,
                acc):
    c = lax.axis_index("c")
    s = lax.axis_index("s")
    r0 = s * _RPT
    pltpu.sync_copy(zero_hbm.at[pl.ds(0, _RPT)], acc.at[pl.ds(r0, _RPT)])
    plsc.subcore_barrier()
    base = (c * _NS + s) * _EPT

    def body(i, carry):
        e0 = base + i * _C
        pltpu.sync_copy(src_hbm.at[pl.ds(e0, _C)], sidx)
        pltpu.sync_copy(dst_hbm.at[pl.ds(e0, _C)], didx)
        pltpu.sync_copy(g_hbm.at[sidx], rows)
        pltpu.sync_copy(rows, acc.at[didx], add=True)
        return carry

    lax.fori_loop(0, _NCHUNK, body, 0)
    plsc.subcore_barrier()
    pltpu.sync_copy(acc.at[pl.ds(r0, _RPT)], out_hbm.at[c, pl.ds(r0, _RPT)])


def _agg_partials(g, src, dst, zeros_rows):
    return pl.kernel(
        _agg_kernel,
        mesh=_SC_MESH,
        out_type=jax.ShapeDtypeStruct((_NC, _N, _D), jnp.float32),
        scratch_types=[
            pltpu.VMEM((_C,), jnp.int32),
            pltpu.VMEM((_C,), jnp.int32),
            pltpu.VMEM((_C, _D), jnp.float32),
            pltpu.VMEM_SHARED((_N, _D), jnp.float32),
        ],
    )(g, src, dst, zeros_rows)


_BN = 1000


def _tc_first_body(d0_ref, d1_ref, x_ref, w_ref, g_ref, dis_ref):
    deg = d0_ref[:, :1] + d1_ref[:, :1] + 1.0
    dis = lax.rsqrt(deg)
    h = jnp.dot(x_ref[...], w_ref[...], preferred_element_type=jnp.float32)
    g_ref[...] = h * dis
    dis_ref[...] = dis


def _tc_first(deg0, deg1, x, W1):
    grid = (_N // _BN,)
    return pl.pallas_call(
        _tc_first_body,
        grid=grid,
        in_specs=[
            pl.BlockSpec((_BN, 16), lambda i: (i, 0)),
            pl.BlockSpec((_BN, 16), lambda i: (i, 0)),
            pl.BlockSpec((_BN, _D), lambda i: (i, 0)),
            pl.BlockSpec((_D, _D), lambda i: (0, 0)),
        ],
        out_specs=[
            pl.BlockSpec((_BN, _D), lambda i: (i, 0)),
            pl.BlockSpec((_BN, 1), lambda i: (i, 0)),
        ],
        out_shape=[
            jax.ShapeDtypeStruct((_N, _D), jnp.float32),
            jax.ShapeDtypeStruct((_N, 1), jnp.float32),
        ],
    )(deg0, deg1, x, W1)


def _tc_mid_body(p0_ref, p1_ref, g_ref, dis_ref, w_ref, b_ref, out_ref):
    dis = dis_ref[...]
    t = (p0_ref[...] + p1_ref[...] + g_ref[...]) * dis + b_ref[...]
    t = jnp.maximum(t, 0.0)
    out_ref[...] = jnp.dot(t, w_ref[...],
                           preferred_element_type=jnp.float32) * dis


def _tc_mid(p0, p1, g, dis, W2, b1):
    grid = (_N // _BN,)
    return pl.pallas_call(
        _tc_mid_body,
        grid=grid,
        in_specs=[
            pl.BlockSpec((_BN, _D), lambda i: (i, 0)),
            pl.BlockSpec((_BN, _D), lambda i: (i, 0)),
            pl.BlockSpec((_BN, _D), lambda i: (i, 0)),
            pl.BlockSpec((_BN, 1), lambda i: (i, 0)),
            pl.BlockSpec((_D, _D), lambda i: (0, 0)),
            pl.BlockSpec((1, _D), lambda i: (0, 0)),
        ],
        out_specs=pl.BlockSpec((_BN, _D), lambda i: (i, 0)),
        out_shape=jax.ShapeDtypeStruct((_N, _D), jnp.float32),
    )(p0, p1, g, dis, W2, b1)


def _tc_head_body(p0_ref, p1_ref, g_ref, dis_ref, b_ref, sv_ref, ac_ref,
                  w1a_ref, w1b_ref, w1c_ref, b1_ref, w2_ref, b2_ref, w3_ref,
                  b3_ref, out_ref, acc_ref):
    i = pl.program_id(0)

    @pl.when(i == 0)
    def _():
        acc_ref[...] = jnp.zeros_like(acc_ref)

    t = (p0_ref[...] + p1_ref[...] + g_ref[...]) * dis_ref[...] + b_ref[...]
    t = jnp.maximum(t, 0.0)
    acc_ref[...] += jnp.sum(t, axis=0, keepdims=True)

    @pl.when(i == (_N // _BN) - 1)
    def _():
        pooled = acc_ref[...] * (1.0 / float(_N))
        z = (jnp.dot(pooled, w1a_ref[...], preferred_element_type=jnp.float32)
             + jnp.dot(sv_ref[...], w1b_ref[...],
                       preferred_element_type=jnp.float32)
             + jnp.dot(ac_ref[...], w1c_ref[...],
                       preferred_element_type=jnp.float32)
             + b1_ref[...])
        z = jnp.maximum(z, 0.0)
        z = jnp.dot(z, w2_ref[...], preferred_element_type=jnp.float32)
        z = jnp.maximum(z + b2_ref[...], 0.0)
        out_ref[...] = (jnp.dot(z, w3_ref[...],
                                preferred_element_type=jnp.float32)
                        + b3_ref[...])


def _tc_head(p0, p1, g, dis, b2, sv, ac, fW1a, fW1b, fW1c, fb1, fW2, fb2,
             fW3, fb3):
    grid = (_N // _BN,)

    def full(shape):
        return pl.BlockSpec(shape, lambda i: tuple(0 for _ in shape))

    return pl.pallas_call(
        _tc_head_body,
        grid=grid,
        in_specs=[
            pl.BlockSpec((_BN, _D), lambda i: (i, 0)),
            pl.BlockSpec((_BN, _D), lambda i: (i, 0)),
            pl.BlockSpec((_BN, _D), lambda i: (i, 0)),
            pl.BlockSpec((_BN, 1), lambda i: (i, 0)),
            full((1, _D)),
            full((1, 64)),
            full((1, 16)),
            full((_D, 256)),
            full((64, 256)),
            full((16, 256)),
            full((1, 256)),
            full((256, 256)),
            full((1, 256)),
            full((256, 1)),
            full((1, 1)),
        ],
        out_specs=pl.BlockSpec((1, 1), lambda i: (0, 0)),
        out_shape=jax.ShapeDtypeStruct((1, 1), jnp.float32),
        scratch_shapes=[pltpu.VMEM((1, _D), jnp.float32)],
    )(p0, p1, g, dis, b2, sv, ac, fW1a, fW1b, fW1c, fb1, fW2, fb2, fW3, fb3)


@jax.jit
def kernel(x, edge_index, batch, state_vector, action, W1, b1, W2, b2, fW1,
           fb1, fW2, fb2, fW3, fb3):
    src = edge_index[0]
    dst = edge_index[1]
    ones16 = jnp.ones((_C, 16), jnp.float32)
    zeros16 = jnp.zeros((_RPT, 16), jnp.float32)
    zeros_rows = jnp.zeros((_RPT, _D), jnp.float32)

    degp = _deg_partials(dst, ones16, zeros16)
    g1, dis = _tc_first(degp[0], degp[1], x, W1)
    p1 = _agg_partials(g1, src, dst, zeros_rows)
    g2 = _tc_mid(p1[0], p1[1], g1, dis, W2, b1.reshape(1, _D))
    p2 = _agg_partials(g2, src, dst, zeros_rows)
    out = _tc_head(p2[0], p2[1], g2, dis, b2.reshape(1, _D), state_vector,
                   action, fW1[:_D], fW1[_D:_D + 64], fW1[_D + 64:],
                   fb1.reshape(1, 256), fW2, fb2.reshape(1, 256), fW3,
                   fb3.reshape(1, 1))
    return out


# speedup vs baseline: 13.0035x; 13.0035x over previous
import functools

import jax
import jax.numpy as jnp
from jax import lax
from jax.experimental import pallas as pl
from jax.experimental.pallas import tpu as pltpu
from jax.experimental.pallas import tpu_sc as plsc

_N = 10000
_E = 320000
_D = 128
_NC = 2
_NS = 16
_NW = _NC * _NS
_EPT = _E // _NW
_C = 80
_NCHUNK = _EPT // _C
_RPT = 624
_TAIL0 = _RPT * _NS
_TAILN = _N - _TAIL0

_SC_MESH = plsc.VectorSubcoreMesh(core_axis_name="c", subcore_axis_name="s")


def _deg_kernel(dst_hbm, zero_hbm, out_hbm, didx, cnt):
    c = lax.axis_index("c")
    s = lax.axis_index("s")
    w = c * _NS + s
    pltpu.sync_copy(zero_hbm, cnt)
    base = w * _EPT
    ones_v = jnp.ones((16,), jnp.float32)

    def body(i, carry):
        e0 = base + i * _C
        pltpu.sync_copy(dst_hbm.at[pl.ds(e0, _C)], didx)
        for k in range(_C // 16):
            idx16 = didx[pl.ds(k * 16, 16)]
            plsc.addupdate_scatter(cnt, [idx16], ones_v)
        return carry

    lax.fori_loop(0, _NCHUNK, body, 0)
    pltpu.sync_copy(cnt, out_hbm.at[pl.ds(w * _N, _N)])


def _deg_partials(dst, zeros_n):
    return pl.kernel(
        _deg_kernel,
        mesh=_SC_MESH,
        out_type=jax.ShapeDtypeStruct((_NW * _N,), jnp.float32),
        scratch_types=[
            pltpu.VMEM((_C,), jnp.int32),
            pltpu.VMEM((_N,), jnp.float32),
        ],
        compiler_params=pltpu.CompilerParams(needs_layout_passes=False),
    )(dst, zeros_n)


def _agg_kernel(g_hbm, src_hbm, dst_hbm, zero_hbm, out_hbm, sidx, didx, rows,
                acc):
    c = lax.axis_index("c")
    s = lax.axis_index("s")
    r0 = s * _RPT
    last = s == _NS - 1
    pltpu.sync_copy(zero_hbm.at[pl.ds(0, _RPT)], acc.at[pl.ds(r0, _RPT)])

    @pl.when(last)
    def _():
        pltpu.sync_copy(zero_hbm.at[pl.ds(0, _TAILN)],
                        acc.at[pl.ds(_TAIL0, _TAILN)])

    plsc.subcore_barrier()
    base = (c * _NS + s) * _EPT

    def body(i, carry):
        e0 = base + i * _C
        pltpu.sync_copy(src_hbm.at[pl.ds(e0, _C)], sidx)
        pltpu.sync_copy(dst_hbm.at[pl.ds(e0, _C)], didx)
        pltpu.sync_copy(g_hbm.at[sidx], rows)
        pltpu.sync_copy(rows, acc.at[didx], add=True)
        return carry

    lax.fori_loop(0, _NCHUNK, body, 0)
    plsc.subcore_barrier()
    pltpu.sync_copy(acc.at[pl.ds(r0, _RPT)], out_hbm.at[c, pl.ds(r0, _RPT)])

    @pl.when(last)
    def _():
        pltpu.sync_copy(acc.at[pl.ds(_TAIL0, _TAILN)],
                        out_hbm.at[c, pl.ds(_TAIL0, _TAILN)])


def _agg_partials(g, src, dst, zeros_rows):
    return pl.kernel(
        _agg_kernel,
        mesh=_SC_MESH,
        out_type=jax.ShapeDtypeStruct((_NC, _N, _D), jnp.float32),
        scratch_types=[
            pltpu.VMEM((_C,), jnp.int32),
            pltpu.VMEM((_C,), jnp.int32),
            pltpu.VMEM((_C, _D), jnp.float32),
            pltpu.VMEM_SHARED((_N, _D), jnp.float32),
        ],
    )(g, src, dst, zeros_rows)


_BN = 1000


def _tc_first_body(dt_ref, x_ref, w_ref, g_ref, dis_ref):
    deg = jnp.sum(dt_ref[...], axis=1, keepdims=True) + 1.0
    dis = lax.rsqrt(deg)
    h = jnp.dot(x_ref[...], w_ref[...], preferred_element_type=jnp.float32)
    g_ref[...] = h * dis
    dis_ref[...] = dis


def _tc_first(degT, x, W1):
    grid = (_N // _BN,)
    return pl.pallas_call(
        _tc_first_body,
        grid=grid,
        in_specs=[
            pl.BlockSpec((_BN, _NW), lambda i: (i, 0)),
            pl.BlockSpec((_BN, _D), lambda i: (i, 0)),
            pl.BlockSpec((_D, _D), lambda i: (0, 0)),
        ],
        out_specs=[
            pl.BlockSpec((_BN, _D), lambda i: (i, 0)),
            pl.BlockSpec((_BN, 1), lambda i: (i, 0)),
        ],
        out_shape=[
            jax.ShapeDtypeStruct((_N, _D), jnp.float32),
            jax.ShapeDtypeStruct((_N, 1), jnp.float32),
        ],
    )(degT, x, W1)


def _tc_mid_body(p0_ref, p1_ref, g_ref, dis_ref, w_ref, b_ref, out_ref):
    dis = dis_ref[...]
    t = (p0_ref[...] + p1_ref[...] + g_ref[...]) * dis + b_ref[...]
    t = jnp.maximum(t, 0.0)
    out_ref[...] = jnp.dot(t, w_ref[...],
                           preferred_element_type=jnp.float32) * dis


def _tc_mid(p0, p1, g, dis, W2, b1):
    grid = (_N // _BN,)
    return pl.pallas_call(
        _tc_mid_body,
        grid=grid,
        in_specs=[
            pl.BlockSpec((_BN, _D), lambda i: (i, 0)),
            pl.BlockSpec((_BN, _D), lambda i: (i, 0)),
            pl.BlockSpec((_BN, _D), lambda i: (i, 0)),
            pl.BlockSpec((_BN, 1), lambda i: (i, 0)),
            pl.BlockSpec((_D, _D), lambda i: (0, 0)),
            pl.BlockSpec((1, _D), lambda i: (0, 0)),
        ],
        out_specs=pl.BlockSpec((_BN, _D), lambda i: (i, 0)),
        out_shape=jax.ShapeDtypeStruct((_N, _D), jnp.float32),
    )(p0, p1, g, dis, W2, b1)


def _tc_head_body(p0_ref, p1_ref, g_ref, dis_ref, b_ref, sv_ref, ac_ref,
                  w1a_ref, w1b_ref, w1c_ref, b1_ref, w2_ref, b2_ref, w3_ref,
                  b3_ref, out_ref, acc_ref):
    i = pl.program_id(0)

    @pl.when(i == 0)
    def _():
        acc_ref[...] = jnp.zeros_like(acc_ref)

    t = (p0_ref[...] + p1_ref[...] + g_ref[...]) * dis_ref[...] + b_ref[...]
    t = jnp.maximum(t, 0.0)
    acc_ref[...] += jnp.sum(t, axis=0, keepdims=True)

    @pl.when(i == (_N // _BN) - 1)
    def _():
        pooled = acc_ref[...] * (1.0 / float(_N))
        z = (jnp.dot(pooled, w1a_ref[...], preferred_element_type=jnp.float32)
             + jnp.dot(sv_ref[...], w1b_ref[...],
                       preferred_element_type=jnp.float32)
             + jnp.dot(ac_ref[...], w1c_ref[...],
                       preferred_element_type=jnp.float32)
             + b1_ref[...])
        z = jnp.maximum(z, 0.0)
        z = jnp.dot(z, w2_ref[...], preferred_element_type=jnp.float32)
        z = jnp.maximum(z + b2_ref[...], 0.0)
        out_ref[...] = (jnp.dot(z, w3_ref[...],
                                preferred_element_type=jnp.float32)
                        + b3_ref[...])


def _tc_head(p0, p1, g, dis, b2, sv, ac, fW1a, fW1b, fW1c, fb1, fW2, fb2,
             fW3, fb3):
    grid = (_N // _BN,)

    def full(shape):
        return pl.BlockSpec(shape, lambda i: tuple(0 for _ in shape))

    return pl.pallas_call(
        _tc_head_body,
        grid=grid,
        in_specs=[
            pl.BlockSpec((_BN, _D), lambda i: (i, 0)),
            pl.BlockSpec((_BN, _D), lambda i: (i, 0)),
            pl.BlockSpec((_BN, _D), lambda i: (i, 0)),
            pl.BlockSpec((_BN, 1), lambda i: (i, 0)),
            full((1, _D)),
            full((1, 64)),
            full((1, 16)),
            full((_D, 256)),
            full((64, 256)),
            full((16, 256)),
            full((1, 256)),
            full((256, 256)),
            full((1, 256)),
            full((256, 1)),
            full((1, 1)),
        ],
        out_specs=pl.BlockSpec((1, 1), lambda i: (0, 0)),
        out_shape=jax.ShapeDtypeStruct((1, 1), jnp.float32),
        scratch_shapes=[pltpu.VMEM((1, _D), jnp.float32)],
    )(p0, p1, g, dis, b2, sv, ac, fW1a, fW1b, fW1c, fb1, fW2, fb2, fW3, fb3)


@jax.jit
def kernel(x, edge_index, batch, state_vector, action, W1, b1, W2, b2, fW1,
           fb1, fW2, fb2, fW3, fb3):
    src = edge_index[0]
    dst = edge_index[1]
    zeros_n = jnp.zeros((_N,), jnp.float32)
    zeros_rows = jnp.zeros((_RPT, _D), jnp.float32)

    degp = _deg_partials(dst, zeros_n)
    degT = degp.reshape(_NW, _N).T
    g1, dis = _tc_first(degT, x, W1)
    p1 = _agg_partials(g1, src, dst, zeros_rows)
    g2 = _tc_mid(p1[0], p1[1], g1, dis, W2, b1.reshape(1, _D))
    p2 = _agg_partials(g2, src, dst, zeros_rows)
    out = _tc_head(p2[0], p2[1], g2, dis, b2.reshape(1, _D), state_vector,
                   action, fW1[:_D], fW1[_D:_D + 64], fW1[_D + 64:],
                   fb1.reshape(1, 256), fW2, fb2.reshape(1, 256), fW3,
                   fb3.reshape(1, 1))
    return out


# speedup vs baseline: 34.9685x; 2.6892x over previous
import functools

import jax
import jax.numpy as jnp
from jax import lax
from jax.experimental import pallas as pl
from jax.experimental.pallas import tpu as pltpu
from jax.experimental.pallas import tpu_sc as plsc

_N = 10000
_E = 320000
_D = 128
_NC = 2
_NS = 16
_NW = _NC * _NS
_EPT = _E // _NW
_C = 40
_NCHUNK = _EPT // _C
_RPT = 624
_TAIL0 = _RPT * _NS
_TAILN = _N - _TAIL0

_SC_MESH = plsc.VectorSubcoreMesh(core_axis_name="c", subcore_axis_name="s")


def _deg_kernel(dst_hbm, zero_hbm, out_hbm, didx_all, cnt):
    c = lax.axis_index("c")
    s = lax.axis_index("s")
    w = c * _NS + s
    pltpu.sync_copy(zero_hbm, cnt)
    pltpu.sync_copy(dst_hbm.at[pl.ds(w * _EPT, _EPT)], didx_all)
    ones_v = jnp.ones((16,), jnp.float32)

    def body(i, carry):
        idx16 = didx_all[pl.ds(i * 16, 16)]
        plsc.addupdate_scatter(cnt, [idx16], ones_v)
        return carry

    lax.fori_loop(0, _EPT // 16, body, 0)
    pltpu.sync_copy(cnt, out_hbm.at[pl.ds(w * _N, _N)])


def _deg_partials(dst, zeros_n):
    return pl.kernel(
        _deg_kernel,
        mesh=_SC_MESH,
        out_type=jax.ShapeDtypeStruct((_NW * _N,), jnp.float32),
        scratch_types=[
            pltpu.VMEM((_EPT,), jnp.int32),
            pltpu.VMEM((_N,), jnp.float32),
        ],
        compiler_params=pltpu.CompilerParams(needs_layout_passes=False),
    )(dst, zeros_n)


_NBUF = 5
_NGRP = _NCHUNK // _NBUF


def _agg_kernel(g_hbm, src_hbm, dst_hbm, zero_hbm, out_hbm, sidx_all, didx,
                rows, acc, zsem, isem, gsem):
    c = lax.axis_index("c")
    s = lax.axis_index("s")
    r0 = s * _RPT
    last = s == _NS - 1
    base = (c * _NS + s) * _EPT

    pltpu.async_copy(zero_hbm.at[pl.ds(0, _RPT)], acc.at[pl.ds(r0, _RPT)],
                     zsem)
    pltpu.sync_copy(src_hbm.at[pl.ds(base, _EPT)], sidx_all)

    def i_start(j, b):
        pltpu.async_copy(dst_hbm.at[pl.ds(base + j * _C, _C)], didx[b],
                         isem[b])

    def i_wait(b):
        pltpu.make_async_copy(dst_hbm.at[pl.ds(0, _C)], didx[b],
                              isem[b]).wait()

    def g_start(j, b):
        pltpu.async_copy(g_hbm.at[sidx_all.at[pl.ds(j * _C, _C)]], rows[b],
                         gsem[b])

    def g_wait(b):
        pltpu.make_async_copy(g_hbm.at[sidx_all.at[pl.ds(0, _C)]], rows[b],
                              gsem[b]).wait()

    for b in range(_NBUF):
        i_start(b, b)
        g_start(b, b)

    pltpu.make_async_copy(zero_hbm.at[pl.ds(0, _RPT)],
                          acc.at[pl.ds(r0, _RPT)], zsem).wait()

    @pl.when(last)
    def _():
        pltpu.sync_copy(zero_hbm.at[pl.ds(0, _TAILN)],
                        acc.at[pl.ds(_TAIL0, _TAILN)])

    plsc.subcore_barrier()

    def body(grp, carry):
        for b in range(_NBUF):
            j = grp * _NBUF + b
            i_wait(b)
            g_wait(b)
            pltpu.sync_copy(rows[b], acc.at[didx[b]], add=True)
            jj = jnp.minimum(j + _NBUF, _NCHUNK - 1)
            i_start(jj, b)
            g_start(jj, b)
        return carry

    lax.fori_loop(0, _NGRP, body, 0)
    for b in range(_NBUF):
        i_wait(b)
        g_wait(b)
    plsc.subcore_barrier()
    pltpu.sync_copy(acc.at[pl.ds(r0, _RPT)], out_hbm.at[c, pl.ds(r0, _RPT)])

    @pl.when(last)
    def _():
        pltpu.sync_copy(acc.at[pl.ds(_TAIL0, _TAILN)],
                        out_hbm.at[c, pl.ds(_TAIL0, _TAILN)])


def _agg_partials(g, src, dst, zeros_rows):
    return pl.kernel(
        _agg_kernel,
        mesh=_SC_MESH,
        out_type=jax.ShapeDtypeStruct((_NC, _N, _D), jnp.float32),
        scratch_types=[
            pltpu.VMEM((_EPT,), jnp.int32),
            [pltpu.VMEM((_C,), jnp.int32) for _ in range(_NBUF)],
            [pltpu.VMEM((_C, _D), jnp.float32) for _ in range(_NBUF)],
            pltpu.VMEM_SHARED((_N, _D), jnp.float32),
            pltpu.SemaphoreType.DMA,
            [pltpu.SemaphoreType.DMA for _ in range(_NBUF)],
            [pltpu.SemaphoreType.DMA for _ in range(_NBUF)],
        ],
        compiler_params=pltpu.CompilerParams(needs_layout_passes=False),
    )(g, src, dst, zeros_rows)


_BN = 1000


def _tc_first_body(dt_ref, x_ref, w_ref, g_ref, dis_ref):
    deg = jnp.sum(dt_ref[...], axis=1, keepdims=True) + 1.0
    dis = lax.rsqrt(deg)
    h = jnp.dot(x_ref[...], w_ref[...], preferred_element_type=jnp.float32)
    g_ref[...] = h * dis
    dis_ref[...] = dis


def _tc_first(degT, x, W1):
    grid = (_N // _BN,)
    return pl.pallas_call(
        _tc_first_body,
        grid=grid,
        in_specs=[
            pl.BlockSpec((_BN, _NW), lambda i: (i, 0)),
            pl.BlockSpec((_BN, _D), lambda i: (i, 0)),
            pl.BlockSpec((_D, _D), lambda i: (0, 0)),
        ],
        out_specs=[
            pl.BlockSpec((_BN, _D), lambda i: (i, 0)),
            pl.BlockSpec((_BN, 1), lambda i: (i, 0)),
        ],
        out_shape=[
            jax.ShapeDtypeStruct((_N, _D), jnp.float32),
            jax.ShapeDtypeStruct((_N, 1), jnp.float32),
        ],
    )(degT, x, W1)


def _tc_mid_body(p0_ref, p1_ref, g_ref, dis_ref, w_ref, b_ref, out_ref):
    dis = dis_ref[...]
    t = (p0_ref[...] + p1_ref[...] + g_ref[...]) * dis + b_ref[...]
    t = jnp.maximum(t, 0.0)
    out_ref[...] = jnp.dot(t, w_ref[...],
                           preferred_element_type=jnp.float32) * dis


def _tc_mid(p0, p1, g, dis, W2, b1):
    grid = (_N // _BN,)
    return pl.pallas_call(
        _tc_mid_body,
        grid=grid,
        in_specs=[
            pl.BlockSpec((_BN, _D), lambda i: (i, 0)),
            pl.BlockSpec((_BN, _D), lambda i: (i, 0)),
            pl.BlockSpec((_BN, _D), lambda i: (i, 0)),
            pl.BlockSpec((_BN, 1), lambda i: (i, 0)),
            pl.BlockSpec((_D, _D), lambda i: (0, 0)),
            pl.BlockSpec((1, _D), lambda i: (0, 0)),
        ],
        out_specs=pl.BlockSpec((_BN, _D), lambda i: (i, 0)),
        out_shape=jax.ShapeDtypeStruct((_N, _D), jnp.float32),
    )(p0, p1, g, dis, W2, b1)


def _tc_head_body(p0_ref, p1_ref, g_ref, dis_ref, b_ref, sv_ref, ac_ref,
                  w1a_ref, w1b_ref, w1c_ref, b1_ref, w2_ref, b2_ref, w3_ref,
                  b3_ref, out_ref, acc_ref):
    i = pl.program_id(0)

    @pl.when(i == 0)
    def _():
        acc_ref[...] = jnp.zeros_like(acc_ref)

    t = (p0_ref[...] + p1_ref[...] + g_ref[...]) * dis_ref[...] + b_ref[...]
    t = jnp.maximum(t, 0.0)
    acc_ref[...] += jnp.sum(t, axis=0, keepdims=True)

    @pl.when(i == (_N // _BN) - 1)
    def _():
        pooled = acc_ref[...] * (1.0 / float(_N))
        z = (jnp.dot(pooled, w1a_ref[...], preferred_element_type=jnp.float32)
             + jnp.dot(sv_ref[...], w1b_ref[...],
                       preferred_element_type=jnp.float32)
             + jnp.dot(ac_ref[...], w1c_ref[...],
                       preferred_element_type=jnp.float32)
             + b1_ref[...])
        z = jnp.maximum(z, 0.0)
        z = jnp.dot(z, w2_ref[...], preferred_element_type=jnp.float32)
        z = jnp.maximum(z + b2_ref[...], 0.0)
        out_ref[...] = (jnp.dot(z, w3_ref[...],
                                preferred_element_type=jnp.float32)
                        + b3_ref[...])


def _tc_head(p0, p1, g, dis, b2, sv, ac, fW1a, fW1b, fW1c, fb1, fW2, fb2,
             fW3, fb3):
    grid = (_N // _BN,)

    def full(shape):
        return pl.BlockSpec(shape, lambda i: tuple(0 for _ in shape))

    return pl.pallas_call(
        _tc_head_body,
        grid=grid,
        in_specs=[
            pl.BlockSpec((_BN, _D), lambda i: (i, 0)),
            pl.BlockSpec((_BN, _D), lambda i: (i, 0)),
            pl.BlockSpec((_BN, _D), lambda i: (i, 0)),
            pl.BlockSpec((_BN, 1), lambda i: (i, 0)),
            full((1, _D)),
            full((1, 64)),
            full((1, 16)),
            full((_D, 256)),
            full((64, 256)),
            full((16, 256)),
            full((1, 256)),
            full((256, 256)),
            full((1, 256)),
            full((256, 1)),
            full((1, 1)),
        ],
        out_specs=pl.BlockSpec((1, 1), lambda i: (0, 0)),
        out_shape=jax.ShapeDtypeStruct((1, 1), jnp.float32),
        scratch_shapes=[pltpu.VMEM((1, _D), jnp.float32)],
    )(p0, p1, g, dis, b2, sv, ac, fW1a, fW1b, fW1c, fb1, fW2, fb2, fW3, fb3)


@jax.jit
def kernel(x, edge_index, batch, state_vector, action, W1, b1, W2, b2, fW1,
           fb1, fW2, fb2, fW3, fb3):
    src = edge_index[0]
    dst = edge_index[1]
    zeros_n = jnp.zeros((_N,), jnp.float32)
    zeros_rows = jnp.zeros((_RPT, _D), jnp.float32)

    degp = _deg_partials(dst, zeros_n)
    degT = degp.reshape(_NW, _N).T
    g1, dis = _tc_first(degT, x, W1)
    p1 = _agg_partials(g1, src, dst, zeros_rows)
    g2 = _tc_mid(p1[0], p1[1], g1, dis, W2, b1.reshape(1, _D))
    p2 = _agg_partials(g2, src, dst, zeros_rows)
    out = _tc_head(p2[0], p2[1], g2, dis, b2.reshape(1, _D), state_vector,
                   action, fW1[:_D], fW1[_D:_D + 64], fW1[_D + 64:],
                   fb1.reshape(1, 256), fW2, fb2.reshape(1, 256), fW3,
                   fb3.reshape(1, 1))
    return out
